# baseline (speedup 1.0000x reference)
import jax
import jax.numpy as jnp
import numpy as np
from jax.experimental import pallas as pl

B = 32
L = 8192
NUM_CLASSES = 7
TOP_K = 200
CONF_THRESH = 0.01
NMS_THRESH = 0.1
VAR0, VAR1 = 0.1, 0.2
PRIORS_PER_LOC = 3
STEM = [(1,64,7,2,3),(64,64,3,2,1),(64,64,3,1,1),(64,128,3,2,1),(128,128,3,1,1),(128,128,3,2,1)]
STAGES = [(128,128,3,1,1),(128,128,3,2,1),(128,128,3,2,1),(128,128,3,2,1),(128,128,3,2,1),(128,128,3,2,1)]
FEAT_LENS = [512,256,128,64,32,16]


def _conv1d(x, w, b, stride, pad):
    y = jax.lax.conv_general_dilated(x, w, (stride,), [(pad, pad)], dimension_numbers=("NCH", "OIH", "NCH"))
    return y + b[None, :, None]


def _priors():
    pr = []
    for f in FEAT_LENS:
        cx = np.repeat((np.arange(f) + 0.5) / f, PRIORS_PER_LOC)
        w = np.tile(np.array([1.0, 2.0, 3.0]) / f, f)
        pr.append(np.stack([cx, w], 1))
    return jnp.asarray(np.clip(np.concatenate(pr, 0), 0.0, 1.0), jnp.float32)


def _forward_heads(x1, x2, x3, params):
    specs = STEM + STAGES
    xs = [x1, x2, x3]
    branches = [params["res"], params["res2"], params["res3"]]
    sources = []
    for i, (ci, co, k, s, p) in enumerate(specs):
        xs = [jax.nn.relu(_conv1d(x, br[i]["w"], br[i]["b"], s, p)) for x, br in zip(xs, branches)]
        if i >= 6:
            sources.append(jnp.concatenate(xs, 1))
    locs, confs = [], []
    for j, src in enumerate(sources):
        lo = _conv1d(src, params["loc"][j]["w"], params["loc"][j]["b"], 1, 1)
        cf = _conv1d(src, params["conf"][j]["w"], params["conf"][j]["b"], 1, 1)
        locs.append(jnp.transpose(lo, (0, 2, 1)).reshape(lo.shape[0], -1))
        confs.append(jnp.transpose(cf, (0, 2, 1)).reshape(cf.shape[0], -1))
    nb = x1.shape[0]
    loc = jnp.concatenate(locs, 1).reshape(nb, -1, 2)
    conf = jax.nn.softmax(jnp.concatenate(confs, 1).reshape(nb, -1, NUM_CLASSES), axis=-1)
    return loc, conf


def _decode_pallas(loc, priors):
    nb, P, _ = loc.shape
    l0 = loc[:, :, 0]
    l1 = loc[:, :, 1]
    pcx = priors[None, :, 0]
    pw = priors[None, :, 1]

    def body(l0_ref, l1_ref, pcx_ref, pw_ref, out_ref):
        cx = pcx_ref[0, :][None, :] + l0_ref[:, :] * (VAR0 * pw_ref[0, :][None, :])
        w = pw_ref[0, :][None, :] * jnp.exp(l1_ref[:, :] * VAR1)
        out_ref[0] = cx - 0.5 * w
        out_ref[1] = cx + 0.5 * w

    out = pl.pallas_call(
        body,
        out_shape=jax.ShapeDtypeStruct((2, nb, P), jnp.float32),
    )(l0, l1, pcx, pw)
    return jnp.stack([out[0], out[1]], axis=-1)


def _nms_one(scores, boxes):
    scores = jnp.where(scores > CONF_THRESH, scores, 0.0)
    top_s, idx = jax.lax.top_k(scores, TOP_K)
    b = boxes[idx]
    x1, x2 = b[:, 0], b[:, 1]
    inter = jnp.maximum(0.0, jnp.minimum(x2[:, None], x2[None, :]) - jnp.maximum(x1[:, None], x1[None, :]))
    area = x2 - x1
    iou = inter / (area[:, None] + area[None, :] - inter + 1e-9)
    rng = jnp.arange(TOP_K)
    def step(keep, i):
        sup = jnp.any(keep & (rng < i) & (iou[:, i] > NMS_THRESH))
        return keep.at[i].set(keep[i] & jnp.logical_not(sup)), None
    keep, _ = jax.lax.scan(step, top_s > 0.0, rng)
    return jnp.where(keep[:, None], jnp.concatenate([top_s[:, None], b], 1), 0.0)


def kernel(x1, x2, x3, params):
    loc, conf = _forward_heads(x1, x2, x3, params)
    boxes = _decode_pallas(loc, _priors())
    cls_scores = jnp.transpose(conf[..., 1:], (0, 2, 1))
    out = jax.vmap(jax.vmap(_nms_one, in_axes=(0, None)), in_axes=(0, 0))(cls_scores, boxes)
    bg = jnp.zeros(out[:, :1].shape, out.dtype)
    return jnp.concatenate([bg, out], 1)


# speedup vs baseline: 2.3182x; 2.3182x over previous
import functools

import jax
import jax.numpy as jnp
import numpy as np
from jax.experimental import pallas as pl
from jax.experimental.pallas import tpu as pltpu

B = 32
L = 8192
NUM_CLASSES = 7
TOP_K = 200
CONF_THRESH = 0.01
NMS_THRESH = 0.1
VAR0, VAR1 = 0.1, 0.2
PRIORS_PER_LOC = 3
STEM = [(1,64,7,2,3),(64,64,3,2,1),(64,64,3,1,1),(64,128,3,2,1),(128,128,3,1,1),(128,128,3,2,1)]
STAGES = [(128,128,3,1,1),(128,128,3,2,1),(128,128,3,2,1),(128,128,3,2,1),(128,128,3,2,1),(128,128,3,2,1)]
FEAT_LENS = [512,256,128,64,32,16]
NUM_INST = B * (NUM_CLASSES - 1)
LANES = 128
NUM_GROUPS = (NUM_INST + LANES - 1) // LANES


def _conv1d(x, w, b, stride, pad):
    y = jax.lax.conv_general_dilated(x, w, (stride,), [(pad, pad)], dimension_numbers=("NCH", "OIH", "NCH"))
    return y + b[None, :, None]


def _priors():
    pr = []
    for f in FEAT_LENS:
        cx = np.repeat((np.arange(f) + 0.5) / f, PRIORS_PER_LOC)
        w = np.tile(np.array([1.0, 2.0, 3.0]) / f, f)
        pr.append(np.stack([cx, w], 1))
    return jnp.asarray(np.clip(np.concatenate(pr, 0), 0.0, 1.0), jnp.float32)


def _forward_heads(x1, x2, x3, params):
    specs = STEM + STAGES
    xs = [x1, x2, x3]
    branches = [params["res"], params["res2"], params["res3"]]
    sources = []
    for i, (ci, co, k, s, p) in enumerate(specs):
        xs = [jax.nn.relu(_conv1d(x, br[i]["w"], br[i]["b"], s, p)) for x, br in zip(xs, branches)]
        if i >= 6:
            sources.append(jnp.concatenate(xs, 1))
    locs, confs = [], []
    for j, src in enumerate(sources):
        lo = _conv1d(src, params["loc"][j]["w"], params["loc"][j]["b"], 1, 1)
        cf = _conv1d(src, params["conf"][j]["w"], params["conf"][j]["b"], 1, 1)
        locs.append(jnp.transpose(lo, (0, 2, 1)).reshape(lo.shape[0], -1))
        confs.append(jnp.transpose(cf, (0, 2, 1)).reshape(cf.shape[0], -1))
    nb = x1.shape[0]
    loc = jnp.concatenate(locs, 1).reshape(nb, -1, 2)
    conf = jax.nn.softmax(jnp.concatenate(confs, 1).reshape(nb, -1, NUM_CLASSES), axis=-1)
    return loc, conf


def _decode_pallas(loc, priors):
    nb, P, _ = loc.shape
    l0 = loc[:, :, 0]
    l1 = loc[:, :, 1]
    pcx = priors[None, :, 0]
    pw = priors[None, :, 1]

    def body(l0_ref, l1_ref, pcx_ref, pw_ref, out_ref):
        cx = pcx_ref[0, :][None, :] + l0_ref[:, :] * (VAR0 * pw_ref[0, :][None, :])
        w = pw_ref[0, :][None, :] * jnp.exp(l1_ref[:, :] * VAR1)
        out_ref[0] = cx - 0.5 * w
        out_ref[1] = cx + 0.5 * w

    out = pl.pallas_call(
        body,
        out_shape=jax.ShapeDtypeStruct((2, nb, P), jnp.float32),
    )(l0, l1, pcx, pw)
    return out[0], out[1]


def _nms_body(s_ref, x1_ref, x2_ref, out_ref, sup_ref):
    s = s_ref[0]
    bx1 = x1_ref[0]
    bx2 = x2_ref[0]
    area = bx2 - bx1
    sup_ref[:, :] = jnp.zeros((TOP_K, LANES), jnp.float32)

    def step(i, _):
        s_i = s_ref[0, pl.ds(i, 1), :]
        sup_i = sup_ref[pl.ds(i, 1), :]
        keep_i = (s_i > 0.0) & (sup_i == 0.0)
        x1_i = x1_ref[0, pl.ds(i, 1), :]
        x2_i = x2_ref[0, pl.ds(i, 1), :]
        area_i = x2_i - x1_i
        inter = jnp.maximum(0.0, jnp.minimum(x2_i, bx2) - jnp.maximum(x1_i, bx1))
        iou = inter / (area_i + area - inter + 1e-9)
        hit = keep_i & (iou > NMS_THRESH)
        sup_ref[:, :] = jnp.where(hit, 1.0, sup_ref[:, :])
        keep_f = jnp.where(keep_i, 1.0, 0.0)
        out_ref[0, 0, pl.ds(i, 1), :] = keep_f * s_i
        out_ref[0, 1, pl.ds(i, 1), :] = keep_f * x1_i
        out_ref[0, 2, pl.ds(i, 1), :] = keep_f * x2_i
        return 0

    jax.lax.fori_loop(0, TOP_K, step, 0)


def _nms_pallas(top_s, gx1, gx2):
    pad = NUM_GROUPS * LANES - NUM_INST

    def prep(a):
        a = jnp.pad(a.T, ((0, 0), (0, pad)))
        return a.reshape(TOP_K, NUM_GROUPS, LANES).transpose(1, 0, 2)

    s_t, x1_t, x2_t = prep(top_s), prep(gx1), prep(gx2)
    out = pl.pallas_call(
        _nms_body,
        grid=(NUM_GROUPS,),
        in_specs=[pl.BlockSpec((1, TOP_K, LANES), lambda g: (g, 0, 0))] * 3,
        out_specs=pl.BlockSpec((1, 3, TOP_K, LANES), lambda g: (g, 0, 0, 0)),
        out_shape=jax.ShapeDtypeStruct((NUM_GROUPS, 3, TOP_K, LANES), jnp.float32),
        scratch_shapes=[pltpu.VMEM((TOP_K, LANES), jnp.float32)],
        compiler_params=pltpu.CompilerParams(
            dimension_semantics=("parallel",),
        ),
    )(s_t, x1_t, x2_t)
    out = out.transpose(0, 3, 2, 1).reshape(NUM_GROUPS * LANES, TOP_K, 3)
    return out[:NUM_INST]


def kernel(x1, x2, x3, params):
    loc, conf = _forward_heads(x1, x2, x3, params)
    bx1, bx2 = _decode_pallas(loc, _priors())
    cls = jnp.transpose(conf[..., 1:], (0, 2, 1))
    cls = jnp.where(cls > CONF_THRESH, cls, 0.0)
    top_s, idx = jax.lax.top_k(cls.reshape(NUM_INST, -1), TOP_K)
    idx3 = idx.reshape(B, (NUM_CLASSES - 1) * TOP_K)
    gx1 = jnp.take_along_axis(bx1, idx3, axis=1).reshape(NUM_INST, TOP_K)
    gx2 = jnp.take_along_axis(bx2, idx3, axis=1).reshape(NUM_INST, TOP_K)
    out = _nms_pallas(top_s, gx1, gx2)
    out = out.reshape(B, NUM_CLASSES - 1, TOP_K, 3)
    bg = jnp.zeros((B, 1, TOP_K, 3), out.dtype)
    return jnp.concatenate([bg, out], 1)
